# Initial kernel scaffold; baseline (speedup 1.0000x reference)
import jax
import jax.numpy as jnp
from jax.experimental import pallas as pl


def kernel(x_user, x_item, edge_index_user_item, edge_index_item_user, W_src, W_dst, att_src, att_dst, bias):
    raise NotImplementedError("write your pallas kernel here")


# speedup vs baseline: 13.1935x; 13.1935x over previous
import functools

import jax
import jax.numpy as jnp
from jax import lax
from jax.experimental import pallas as pl
from jax.experimental.pallas import tpu as pltpu
from jax.experimental.pallas import tpu_sc as plsc

N = 10000
NPAD = 10240
D = 128
E = 160000
NCORE = 2
NSUB = 16
CH = 80
EPT = E // NSUB
NCHUNK = EPT // CH
RPT = NPAD // NSUB
BLK = 1024
NEG = -1e30


def _proj_body(x_ref, ws_ref, avs_ref, wd_ref, avd_ref, h_ref, aa_ref, mx_ref):
    x = x_ref[...]
    dn = (((1,), (1,)), ((), ()))
    ws_row = lax.dot_general(avs_ref[...], ws_ref[...], dn,
                             preferred_element_type=jnp.float32)
    wd_row = lax.dot_general(avd_ref[...], wd_ref[...], dn,
                             preferred_element_type=jnp.float32)
    h_ref[...] = jnp.dot(x, ws_ref[...], preferred_element_type=jnp.float32)
    a_s = jnp.sum(x * ws_row, axis=1, keepdims=True)
    a_d = jnp.sum(x * wd_row, axis=1, keepdims=True)
    aa_ref[...] = jnp.concatenate(
        [a_s, a_d, jnp.zeros((x.shape[0], 14), jnp.float32)], axis=1)
    cur = jnp.concatenate([
        jnp.full((1, 128), jnp.max(a_s), jnp.float32),
        jnp.full((1, 128), jnp.max(a_d), jnp.float32),
        jnp.full((6, 128), NEG, jnp.float32),
    ], axis=0)

    @pl.when(pl.program_id(0) == 0)
    def _():
        mx_ref[...] = cur

    @pl.when(pl.program_id(0) != 0)
    def _():
        mx_ref[...] = jnp.maximum(mx_ref[...], cur)


def _tc_proj(x, Ws, avs, Wd, avd):
    return pl.pallas_call(
        _proj_body,
        grid=(NPAD // BLK,),
        in_specs=[
            pl.BlockSpec((BLK, D), lambda i: (i, 0)),
            pl.BlockSpec((D, D), lambda i: (0, 0)),
            pl.BlockSpec((1, D), lambda i: (0, 0)),
            pl.BlockSpec((D, D), lambda i: (0, 0)),
            pl.BlockSpec((1, D), lambda i: (0, 0)),
        ],
        out_specs=[
            pl.BlockSpec((BLK, D), lambda i: (i, 0)),
            pl.BlockSpec((BLK, 16), lambda i: (i, 0)),
            pl.BlockSpec((8, 128), lambda i: (0, 0)),
        ],
        out_shape=[
            jax.ShapeDtypeStruct((NPAD, D), jnp.float32),
            jax.ShapeDtypeStruct((NPAD, 16), jnp.float32),
            jax.ShapeDtypeStruct((8, 128), jnp.float32),
        ],
    )(x, Ws, avs.reshape(1, D), Wd, avd.reshape(1, D))


def _sc_edge(H, AS, AD, srcg, dstl, g2, b2, zf, zv):
    mesh = plsc.VectorSubcoreMesh(core_axis_name="c", subcore_axis_name="s",
                                  num_cores=NCORE, num_subcores=NSUB)

    @functools.partial(
        pl.kernel,
        out_type=jax.ShapeDtypeStruct((NCORE * NPAD, D), jnp.float32),
        mesh=mesh,
        compiler_params=pltpu.CompilerParams(needs_layout_passes=False),
        scratch_types=[
            pltpu.VMEM_SHARED((NPAD, D), jnp.float32),
            pltpu.VMEM_SHARED((NPAD,), jnp.float32),
            pltpu.VMEM((CH,), jnp.int32),
            pltpu.VMEM((CH,), jnp.int32),
            pltpu.VMEM((CH,), jnp.int32),
            pltpu.VMEM((CH,), jnp.float32),
            pltpu.VMEM((CH,), jnp.float32),
            pltpu.VMEM((CH,), jnp.float32),
            pltpu.VMEM((CH, D), jnp.float32),
            pltpu.VMEM((16,), jnp.float32),
            pltpu.VMEM((D,), jnp.float32),
            pltpu.VMEM((RPT,), jnp.float32),
        ],
    )
    def k(h_hbm, as_hbm, ad_hbm, src_hbm, dst_hbm, g_hbm, b_hbm, zf_hbm,
          zv_hbm, out, acc, dacc, idx_s, idx_d, idx_dg, asv, adv, pv, rows,
          gv, bv, dnv):
        c = lax.axis_index("c")
        s = lax.axis_index("s")
        ro = s * RPT
        pltpu.sync_copy(zf_hbm.at[pl.ds(ro, RPT)], acc.at[pl.ds(ro, RPT)])
        pltpu.sync_copy(zv_hbm.at[pl.ds(ro, RPT)], dacc.at[pl.ds(ro, RPT)])
        pltpu.sync_copy(g_hbm.at[c], gv)
        pltpu.sync_copy(b_hbm.at[c], bv)
        plsc.subcore_barrier()
        ebase = c * E + s * EPT
        coff = c * NPAD

        def chunk(kk, carry):
            off = pl.multiple_of(ebase + kk * CH, 8)
            pltpu.sync_copy(src_hbm.at[pl.ds(off, CH)], idx_s)
            pltpu.sync_copy(dst_hbm.at[pl.ds(off, CH)], idx_d)
            for j in range(CH // 16):
                sl = pl.ds(16 * j, 16)
                idx_dg[sl] = idx_d[sl] + coff
            pltpu.sync_copy(as_hbm.at[idx_s], asv)
            pltpu.sync_copy(ad_hbm.at[idx_dg], adv)
            pltpu.sync_copy(h_hbm.at[idx_s], rows)
            g = gv[...]
            for j in range(CH // 16):
                sl = pl.ds(16 * j, 16)
                a = asv[sl] + adv[sl]
                a = jnp.where(a >= 0.0, a, a * 0.2)
                pv[sl] = jnp.exp(a - g)

            def srow(e, cc):
                pe = plsc.load_gather(pv, [jnp.full((16,), e, jnp.int32)])
                for j in range(8):
                    sl = pl.ds(16 * j, 16)
                    rows[e, sl] = rows[e, sl] * pe
                return cc

            lax.fori_loop(0, CH, srow, 0)
            pltpu.sync_copy(pv, dacc.at[idx_d], add=True)
            pltpu.sync_copy(rows, acc.at[idx_d], add=True)
            return carry

        lax.fori_loop(0, NCHUNK, chunk, 0)
        plsc.subcore_barrier()
        pltpu.sync_copy(dacc.at[pl.ds(ro, RPT)], dnv)

        def wblock(b, carry):
            rbase = ro + b * CH
            pltpu.sync_copy(acc.at[pl.ds(rbase, CH)], rows)

            def nrow(e, cc):
                d = plsc.load_gather(
                    dnv, [jnp.full((16,), b * CH + e, jnp.int32)])
                q = 1.0 / (d + 1e-16)
                for j in range(8):
                    sl = pl.ds(16 * j, 16)
                    rows[e, sl] = jnp.maximum(rows[e, sl] * q + bv[sl], 0.0)
                return cc

            lax.fori_loop(0, CH, nrow, 0)
            pltpu.sync_copy(rows, out.at[pl.ds(coff + rbase, CH)])
            return carry

        lax.fori_loop(0, RPT // CH, wblock, 0)

    return k(H, AS, AD, srcg, dstl, g2, b2, zf, zv)


def kernel(x_user, x_item, edge_index_user_item, edge_index_item_user,
           W_src, W_dst, att_src, att_dst, bias):
    src0 = edge_index_user_item[0].astype(jnp.int32)
    dst0 = edge_index_user_item[1].astype(jnp.int32)
    src1 = edge_index_item_user[0].astype(jnp.int32)
    dst1 = edge_index_item_user[1].astype(jnp.int32)
    srcg = jnp.concatenate([src0, src1 + NPAD])
    dstl = jnp.concatenate([dst0, dst1])
    pad = ((0, NPAD - N), (0, 0))
    xu = jnp.pad(x_user, pad)
    xi = jnp.pad(x_item, pad)
    zf = jnp.zeros((NPAD, D), jnp.float32)
    zv = jnp.zeros((NPAD,), jnp.float32)
    for l in range(2):
        h_u, aa_u, mx_u = _tc_proj(xu, W_src[l, 0], att_src[l, 0],
                                   W_dst[l, 1], att_dst[l, 1])
        h_i, aa_i, mx_i = _tc_proj(xi, W_src[l, 1], att_src[l, 1],
                                   W_dst[l, 0], att_dst[l, 0])
        g0 = mx_u[0, 0] + mx_i[1, 0]
        g1 = mx_i[0, 0] + mx_u[1, 0]
        g0 = jnp.where(g0 >= 0.0, g0, 0.2 * g0)
        g1 = jnp.where(g1 >= 0.0, g1, 0.2 * g1)
        g2 = jnp.stack([jnp.full((16,), g0), jnp.full((16,), g1)])
        b2 = jnp.stack([bias[l, 0], bias[l, 1]])
        H = jnp.concatenate([h_u, h_i], axis=0)
        AS = jnp.concatenate([aa_u[:, 0], aa_i[:, 0]])
        AD = jnp.concatenate([aa_i[:, 1], aa_u[:, 1]])
        out = _sc_edge(H, AS, AD, srcg, dstl, g2, b2, zf, zv)
        xi = out[:NPAD]
        xu = out[NPAD:]
    return xu[:N], xi[:N]


# speedup vs baseline: 24.3066x; 1.8423x over previous
import functools

import jax
import jax.numpy as jnp
from jax import lax
from jax.experimental import pallas as pl
from jax.experimental.pallas import tpu as pltpu
from jax.experimental.pallas import tpu_sc as plsc

N = 10000
NPAD = 10240
D = 128
E = 160000
NCORE = 2
NSUB = 16
CH = 96
EPT = 10080
NCHUNK = EPT // CH
EPC = NSUB * EPT
RPT = NPAD // NSUB
WB = 32
BLK = 1024
NEG = -1e30


def _proj_body(x_ref, ws_ref, avs_ref, wd_ref, avd_ref, h_ref, aa_ref, mx_ref):
    x = x_ref[...]
    dn = (((1,), (1,)), ((), ()))
    ws_row = lax.dot_general(avs_ref[...], ws_ref[...], dn,
                             preferred_element_type=jnp.float32)
    wd_row = lax.dot_general(avd_ref[...], wd_ref[...], dn,
                             preferred_element_type=jnp.float32)
    h_ref[...] = jnp.dot(x, ws_ref[...], preferred_element_type=jnp.float32)
    a_s = jnp.sum(x * ws_row, axis=1, keepdims=True)
    a_d = jnp.sum(x * wd_row, axis=1, keepdims=True)
    aa_ref[...] = jnp.concatenate(
        [a_s, a_d, jnp.zeros((x.shape[0], 14), jnp.float32)], axis=1)
    cur = jnp.concatenate([
        jnp.full((1, 128), jnp.max(a_s), jnp.float32),
        jnp.full((1, 128), jnp.max(a_d), jnp.float32),
        jnp.full((6, 128), NEG, jnp.float32),
    ], axis=0)

    @pl.when(pl.program_id(0) == 0)
    def _():
        mx_ref[...] = cur

    @pl.when(pl.program_id(0) != 0)
    def _():
        mx_ref[...] = jnp.maximum(mx_ref[...], cur)


def _tc_proj(x, Ws, avs, Wd, avd):
    return pl.pallas_call(
        _proj_body,
        grid=(NPAD // BLK,),
        in_specs=[
            pl.BlockSpec((BLK, D), lambda i: (i, 0)),
            pl.BlockSpec((D, D), lambda i: (0, 0)),
            pl.BlockSpec((1, D), lambda i: (0, 0)),
            pl.BlockSpec((D, D), lambda i: (0, 0)),
            pl.BlockSpec((1, D), lambda i: (0, 0)),
        ],
        out_specs=[
            pl.BlockSpec((BLK, D), lambda i: (i, 0)),
            pl.BlockSpec((BLK, 16), lambda i: (i, 0)),
            pl.BlockSpec((8, 128), lambda i: (0, 0)),
        ],
        out_shape=[
            jax.ShapeDtypeStruct((NPAD, D), jnp.float32),
            jax.ShapeDtypeStruct((NPAD, 16), jnp.float32),
            jax.ShapeDtypeStruct((8, 128), jnp.float32),
        ],
    )(x, Ws, avs.reshape(1, D), Wd, avd.reshape(1, D))


def _sc_edge(H, AS, AD, srcg, dstl, g2, b2, zf, zv):
    mesh = plsc.VectorSubcoreMesh(core_axis_name="c", subcore_axis_name="s",
                                  num_cores=NCORE, num_subcores=NSUB)

    @functools.partial(
        pl.kernel,
        out_type=jax.ShapeDtypeStruct((NCORE * NPAD, D), jnp.float32),
        mesh=mesh,
        compiler_params=pltpu.CompilerParams(needs_layout_passes=False),
        scratch_types=[
            pltpu.VMEM_SHARED((NPAD, D), jnp.float32),
            pltpu.VMEM_SHARED((NPAD,), jnp.float32),
            pltpu.VMEM((3, CH), jnp.int32),
            pltpu.VMEM((3, CH), jnp.int32),
            pltpu.VMEM((3, CH), jnp.int32),
            pltpu.VMEM((3, CH), jnp.float32),
            pltpu.VMEM((3, CH), jnp.float32),
            pltpu.VMEM((3, CH), jnp.float32),
            pltpu.VMEM((3, CH, D), jnp.float32),
            pltpu.VMEM((WB, D), jnp.float32),
            pltpu.VMEM((16,), jnp.float32),
            pltpu.VMEM((D,), jnp.float32),
            pltpu.VMEM((WB,), jnp.float32),
            pltpu.SemaphoreType.DMA,
            pltpu.SemaphoreType.DMA,
            pltpu.SemaphoreType.DMA,
            pltpu.SemaphoreType.DMA,
            pltpu.SemaphoreType.DMA,
            pltpu.SemaphoreType.DMA,
        ],
    )
    def k(h_hbm, as_hbm, ad_hbm, src_hbm, dst_hbm, g_hbm, b_hbm, zf_hbm,
          zv_hbm, out, acc, dacc, idxs, idxd, idxg, asv, adv, pv, rows,
          nbuf, gv, bv, dnv, gs0, gs1, gs2, ss0, ss1, ss2):
        c = lax.axis_index("c")
        s = lax.axis_index("s")
        gsem = (gs0, gs1, gs2)
        ssem = (ss0, ss1, ss2)
        ro = s * RPT
        pltpu.sync_copy(zf_hbm.at[pl.ds(ro, RPT)], acc.at[pl.ds(ro, RPT)])
        pltpu.sync_copy(zv_hbm.at[pl.ds(ro, RPT)], dacc.at[pl.ds(ro, RPT)])
        pltpu.sync_copy(g_hbm.at[c], gv)
        pltpu.sync_copy(b_hbm.at[c], bv)
        plsc.subcore_barrier()
        ebase = c * EPC + s * EPT
        coff = c * NPAD

        def startg(kk, b):
            off = pl.multiple_of(ebase + kk * CH, 8)
            pltpu.sync_copy(src_hbm.at[pl.ds(off, CH)], idxs.at[b])
            pltpu.sync_copy(dst_hbm.at[pl.ds(off, CH)], idxd.at[b])
            for j in range(CH // 16):
                sl = pl.ds(16 * j, 16)
                idxg[b, sl] = idxd[b, sl] + coff
            pltpu.async_copy(as_hbm.at[idxs.at[b]], asv.at[b], gsem[b])
            pltpu.async_copy(ad_hbm.at[idxg.at[b]], adv.at[b], gsem[b])
            pltpu.async_copy(h_hbm.at[idxs.at[b]], rows.at[b], gsem[b])

        def waitg(b):
            pltpu.make_async_copy(as_hbm.at[pl.ds(0, CH)], asv.at[b],
                                  gsem[b]).wait()
            pltpu.make_async_copy(ad_hbm.at[pl.ds(0, CH)], adv.at[b],
                                  gsem[b]).wait()
            pltpu.make_async_copy(h_hbm.at[pl.ds(0, CH)], rows.at[b],
                                  gsem[b]).wait()

        def starts(b):
            pltpu.async_copy(pv.at[b], dacc.at[idxd.at[b]], ssem[b],
                             add=True)
            pltpu.async_copy(rows.at[b], acc.at[idxd.at[b]], ssem[b],
                             add=True)

        def waits(b):
            pltpu.make_async_copy(zv_hbm.at[pl.ds(0, CH)], pv.at[b],
                                  ssem[b]).wait()
            pltpu.make_async_copy(zf_hbm.at[pl.ds(0, CH)], rows.at[b],
                                  ssem[b]).wait()

        startg(0, 0)
        startg(1, 1)

        def ring(g, carry):
            k0 = 3 * g
            for b in range(3):
                kk = k0 + b
                waitg(b)
                gvec = gv[...]
                for j in range(CH // 16):
                    sl = pl.ds(16 * j, 16)
                    a = asv[b, sl] + adv[b, sl]
                    a = jnp.where(a >= 0.0, a, a * 0.2)
                    pv[b, sl] = jnp.exp(a - gvec)

                def srow(e, cc, b=b):
                    pe = plsc.load_gather(pv.at[b],
                                          [jnp.full((16,), e, jnp.int32)])
                    for j in range(8):
                        sl = pl.ds(16 * j, 16)
                        rows[b, e, sl] = rows[b, e, sl] * pe
                    return cc

                lax.fori_loop(0, CH, srow, 0)
                starts(b)
                bn = (b + 2) % 3
                kn = kk + 2

                @pl.when(jnp.logical_and(kn < NCHUNK, kk >= 1))
                def _(bn=bn):
                    waits(bn)

                @pl.when(kn < NCHUNK)
                def _(kn=kn, bn=bn):
                    startg(kn, bn)

            return carry

        lax.fori_loop(0, NCHUNK // 3, ring, 0)
        waits(0)
        waits(1)
        waits(2)
        plsc.subcore_barrier()

        def wblock(b, carry):
            rbase = ro + b * WB
            pltpu.sync_copy(dacc.at[pl.ds(rbase, WB)], dnv)
            pltpu.sync_copy(acc.at[pl.ds(rbase, WB)], nbuf)

            def nrow(e, cc):
                d = plsc.load_gather(
                    dnv, [jnp.full((16,), e, jnp.int32)])
                q = 1.0 / (d + 1e-16)
                for j in range(8):
                    sl = pl.ds(16 * j, 16)
                    nbuf[e, sl] = jnp.maximum(nbuf[e, sl] * q + bv[sl], 0.0)
                return cc

            lax.fori_loop(0, WB, nrow, 0)
            pltpu.sync_copy(nbuf, out.at[pl.ds(coff + rbase, WB)])
            return carry

        lax.fori_loop(0, RPT // WB, wblock, 0)

    return k(H, AS, AD, srcg, dstl, g2, b2, zf, zv)


def kernel(x_user, x_item, edge_index_user_item, edge_index_item_user,
           W_src, W_dst, att_src, att_dst, bias):
    def _pad_edges(arr, fill):
        a = arr.astype(jnp.int32).reshape(NSUB, E // NSUB)
        a = jnp.pad(a, ((0, 0), (0, EPT - E // NSUB)), constant_values=fill)
        return a.reshape(EPC)

    src0 = _pad_edges(edge_index_user_item[0], 0)
    dst0 = _pad_edges(edge_index_user_item[1], NPAD - 1)
    src1 = _pad_edges(edge_index_item_user[0], 0)
    dst1 = _pad_edges(edge_index_item_user[1], NPAD - 1)
    srcg = jnp.concatenate([src0, src1 + NPAD])
    dstl = jnp.concatenate([dst0, dst1])
    pad = ((0, NPAD - N), (0, 0))
    xu = jnp.pad(x_user, pad)
    xi = jnp.pad(x_item, pad)
    zf = jnp.zeros((NPAD, D), jnp.float32)
    zv = jnp.zeros((NPAD,), jnp.float32)
    for l in range(2):
        h_u, aa_u, mx_u = _tc_proj(xu, W_src[l, 0], att_src[l, 0],
                                   W_dst[l, 1], att_dst[l, 1])
        h_i, aa_i, mx_i = _tc_proj(xi, W_src[l, 1], att_src[l, 1],
                                   W_dst[l, 0], att_dst[l, 0])
        g0 = mx_u[0, 0] + mx_i[1, 0]
        g1 = mx_i[0, 0] + mx_u[1, 0]
        g0 = jnp.where(g0 >= 0.0, g0, 0.2 * g0)
        g1 = jnp.where(g1 >= 0.0, g1, 0.2 * g1)
        g2 = jnp.stack([jnp.full((16,), g0), jnp.full((16,), g1)])
        b2 = jnp.stack([bias[l, 0], bias[l, 1]])
        H = jnp.concatenate([h_u, h_i], axis=0)
        AS = jnp.concatenate([aa_u[:, 0], aa_i[:, 0]])
        AD = jnp.concatenate([aa_i[:, 1], aa_u[:, 1]])
        out = _sc_edge(H, AS, AD, srcg, dstl, g2, b2, zf, zv)
        xi = out[:NPAD]
        xu = out[NPAD:]
    return xu[:N], xi[:N]


# speedup vs baseline: 27.3539x; 1.1254x over previous
import functools

import jax
import jax.numpy as jnp
from jax import lax
from jax.experimental import pallas as pl
from jax.experimental.pallas import tpu as pltpu
from jax.experimental.pallas import tpu_sc as plsc

N = 10000
NPAD = 10240
D = 128
E = 160000
NCORE = 2
NSUB = 16
CH = 96
EPT = 10080
NCHUNK = EPT // CH
EPC = NSUB * EPT
RPT = NPAD // NSUB
WB = 32
BLK = 1024
NEG = -1e30


def _proj_body(x_ref, ws_ref, avs_ref, wd_ref, avd_ref, h_ref, aa_ref, mx_ref):
    x = x_ref[...]
    dn = (((1,), (1,)), ((), ()))
    ws_row = lax.dot_general(avs_ref[...], ws_ref[...], dn,
                             preferred_element_type=jnp.float32)
    wd_row = lax.dot_general(avd_ref[...], wd_ref[...], dn,
                             preferred_element_type=jnp.float32)
    h_ref[...] = jnp.dot(x, ws_ref[...], preferred_element_type=jnp.float32)
    a_s = jnp.sum(x * ws_row, axis=1, keepdims=True)
    a_d = jnp.sum(x * wd_row, axis=1, keepdims=True)
    aa_ref[...] = jnp.concatenate(
        [a_s, a_d, jnp.zeros((x.shape[0], 14), jnp.float32)], axis=1)
    cur = jnp.concatenate([
        jnp.full((1, 128), jnp.max(a_s), jnp.float32),
        jnp.full((1, 128), jnp.max(a_d), jnp.float32),
        jnp.full((6, 128), NEG, jnp.float32),
    ], axis=0)

    @pl.when(pl.program_id(0) == 0)
    def _():
        mx_ref[...] = cur

    @pl.when(pl.program_id(0) != 0)
    def _():
        mx_ref[...] = jnp.maximum(mx_ref[...], cur)


def _tc_proj(x, Ws, avs, Wd, avd):
    return pl.pallas_call(
        _proj_body,
        grid=(NPAD // BLK,),
        in_specs=[
            pl.BlockSpec((BLK, D), lambda i: (i, 0)),
            pl.BlockSpec((D, D), lambda i: (0, 0)),
            pl.BlockSpec((1, D), lambda i: (0, 0)),
            pl.BlockSpec((D, D), lambda i: (0, 0)),
            pl.BlockSpec((1, D), lambda i: (0, 0)),
        ],
        out_specs=[
            pl.BlockSpec((BLK, D), lambda i: (i, 0)),
            pl.BlockSpec((BLK, 16), lambda i: (i, 0)),
            pl.BlockSpec((8, 128), lambda i: (0, 0)),
        ],
        out_shape=[
            jax.ShapeDtypeStruct((NPAD, D), jnp.float32),
            jax.ShapeDtypeStruct((NPAD, 16), jnp.float32),
            jax.ShapeDtypeStruct((8, 128), jnp.float32),
        ],
    )(x, Ws, avs.reshape(1, D), Wd, avd.reshape(1, D))


def _sc_edge(H, AS, AD, srcg, dstl, g2, b2, zf, zv):
    mesh = plsc.VectorSubcoreMesh(core_axis_name="c", subcore_axis_name="s",
                                  num_cores=NCORE, num_subcores=NSUB)

    @functools.partial(
        pl.kernel,
        out_type=jax.ShapeDtypeStruct((NCORE * NPAD, D), jnp.float32),
        mesh=mesh,
        compiler_params=pltpu.CompilerParams(needs_layout_passes=False),
        scratch_types=[
            pltpu.VMEM_SHARED((NPAD, D), jnp.float32),
            pltpu.VMEM_SHARED((NPAD,), jnp.float32),
            pltpu.VMEM((3, CH), jnp.int32),
            pltpu.VMEM((3, CH), jnp.int32),
            pltpu.VMEM((3, CH), jnp.int32),
            pltpu.VMEM((3, CH), jnp.float32),
            pltpu.VMEM((3, CH), jnp.float32),
            pltpu.VMEM((3, CH), jnp.float32),
            pltpu.VMEM((3, CH, D), jnp.float32),
            pltpu.VMEM((WB, D), jnp.float32),
            pltpu.VMEM((16,), jnp.float32),
            pltpu.VMEM((D,), jnp.float32),
            pltpu.VMEM((WB,), jnp.float32),
            pltpu.SemaphoreType.DMA,
            pltpu.SemaphoreType.DMA,
            pltpu.SemaphoreType.DMA,
            pltpu.SemaphoreType.DMA,
            pltpu.SemaphoreType.DMA,
            pltpu.SemaphoreType.DMA,
            pltpu.SemaphoreType.DMA,
            pltpu.SemaphoreType.DMA,
            pltpu.SemaphoreType.DMA,
        ],
    )
    def k(h_hbm, as_hbm, ad_hbm, src_hbm, dst_hbm, g_hbm, b_hbm, zf_hbm,
          zv_hbm, out, acc, dacc, idxs, idxd, idxg, asv, adv, pv, rows,
          nbuf, gv, bv, dnv, gs0, gs1, gs2, ss0, ss1, ss2, is0, is1, is2):
        c = lax.axis_index("c")
        s = lax.axis_index("s")
        gsem = (gs0, gs1, gs2)
        ssem = (ss0, ss1, ss2)
        isem = (is0, is1, is2)
        ro = s * RPT
        pltpu.sync_copy(zf_hbm.at[pl.ds(ro, RPT)], acc.at[pl.ds(ro, RPT)])
        pltpu.sync_copy(zv_hbm.at[pl.ds(ro, RPT)], dacc.at[pl.ds(ro, RPT)])
        pltpu.sync_copy(g_hbm.at[c], gv)
        pltpu.sync_copy(b_hbm.at[c], bv)
        plsc.subcore_barrier()
        ebase = c * EPC + s * EPT
        coff = c * NPAD

        def starti(kk, b):
            off = pl.multiple_of(ebase + kk * CH, 8)
            pltpu.async_copy(src_hbm.at[pl.ds(off, CH)], idxs.at[b], isem[b])
            pltpu.async_copy(dst_hbm.at[pl.ds(off, CH)], idxd.at[b], isem[b])

        def waiti(b):
            pltpu.make_async_copy(src_hbm.at[pl.ds(0, CH)], idxs.at[b],
                                  isem[b]).wait()
            pltpu.make_async_copy(dst_hbm.at[pl.ds(0, CH)], idxd.at[b],
                                  isem[b]).wait()

        def startg(b):
            for j in range(CH // 16):
                sl = pl.ds(16 * j, 16)
                idxg[b, sl] = idxd[b, sl] + coff
            pltpu.async_copy(as_hbm.at[idxs.at[b]], asv.at[b], gsem[b])
            pltpu.async_copy(ad_hbm.at[idxg.at[b]], adv.at[b], gsem[b])
            pltpu.async_copy(h_hbm.at[idxs.at[b]], rows.at[b], gsem[b])

        def waitg(b):
            pltpu.make_async_copy(as_hbm.at[pl.ds(0, CH)], asv.at[b],
                                  gsem[b]).wait()
            pltpu.make_async_copy(ad_hbm.at[pl.ds(0, CH)], adv.at[b],
                                  gsem[b]).wait()
            pltpu.make_async_copy(h_hbm.at[pl.ds(0, CH)], rows.at[b],
                                  gsem[b]).wait()

        def starts(b):
            pltpu.async_copy(pv.at[b], dacc.at[idxd.at[b]], ssem[b],
                             add=True)
            pltpu.async_copy(rows.at[b], acc.at[idxd.at[b]], ssem[b],
                             add=True)

        def waits(b):
            pltpu.make_async_copy(zv_hbm.at[pl.ds(0, CH)], pv.at[b],
                                  ssem[b]).wait()
            pltpu.make_async_copy(zf_hbm.at[pl.ds(0, CH)], rows.at[b],
                                  ssem[b]).wait()

        starti(0, 0)
        waiti(0)
        startg(0)
        starti(1, 1)

        def ring(g, carry):
            k0 = 3 * g
            for b in range(3):
                kk = k0 + b
                bg = (b + 1) % 3
                bn = (b + 2) % 3
                waitg(b)

                @pl.when(kk + 1 < NCHUNK)
                def _(bg=bg):
                    waiti(bg)
                    startg(bg)

                gvec = gv[...]
                for j in range(CH // 16):
                    sl = pl.ds(16 * j, 16)
                    a = asv[b, sl] + adv[b, sl]
                    a = jnp.where(a >= 0.0, a, a * 0.2)
                    pv[b, sl] = jnp.exp(a - gvec)

                def srow(e, cc, b=b):
                    pe = plsc.load_gather(pv.at[b],
                                          [jnp.full((16,), e, jnp.int32)])
                    for j in range(8):
                        sl = pl.ds(16 * j, 16)
                        rows[b, e, sl] = rows[b, e, sl] * pe
                    return cc

                lax.fori_loop(0, CH, srow, 0, unroll=4)
                starts(b)
                kn = kk + 2

                @pl.when(kn < NCHUNK)
                def _(kk=kk, kn=kn, bn=bn):
                    @pl.when(kk >= 1)
                    def _():
                        waits(bn)

                    starti(kn, bn)

            return carry

        lax.fori_loop(0, NCHUNK // 3, ring, 0)
        waits(0)
        waits(1)
        waits(2)
        plsc.subcore_barrier()

        def wblock(b, carry):
            rbase = ro + b * WB
            pltpu.sync_copy(dacc.at[pl.ds(rbase, WB)], dnv)
            pltpu.sync_copy(acc.at[pl.ds(rbase, WB)], nbuf)

            def nrow(e, cc):
                d = plsc.load_gather(
                    dnv, [jnp.full((16,), e, jnp.int32)])
                q = 1.0 / (d + 1e-16)
                for j in range(8):
                    sl = pl.ds(16 * j, 16)
                    nbuf[e, sl] = jnp.maximum(nbuf[e, sl] * q + bv[sl], 0.0)
                return cc

            lax.fori_loop(0, WB, nrow, 0, unroll=4)
            pltpu.sync_copy(nbuf, out.at[pl.ds(coff + rbase, WB)])
            return carry

        lax.fori_loop(0, RPT // WB, wblock, 0)

    return k(H, AS, AD, srcg, dstl, g2, b2, zf, zv)


def kernel(x_user, x_item, edge_index_user_item, edge_index_item_user,
           W_src, W_dst, att_src, att_dst, bias):
    def _pad_edges(arr, fill):
        a = arr.astype(jnp.int32).reshape(NSUB, E // NSUB)
        a = jnp.pad(a, ((0, 0), (0, EPT - E // NSUB)), constant_values=fill)
        return a.reshape(EPC)

    src0 = _pad_edges(edge_index_user_item[0], 0)
    dst0 = _pad_edges(edge_index_user_item[1], NPAD - 1)
    src1 = _pad_edges(edge_index_item_user[0], 0)
    dst1 = _pad_edges(edge_index_item_user[1], NPAD - 1)
    srcg = jnp.concatenate([src0, src1 + NPAD])
    dstl = jnp.concatenate([dst0, dst1])
    pad = ((0, NPAD - N), (0, 0))
    xu = jnp.pad(x_user, pad)
    xi = jnp.pad(x_item, pad)
    zf = jnp.zeros((NPAD, D), jnp.float32)
    zv = jnp.zeros((NPAD,), jnp.float32)
    for l in range(2):
        h_u, aa_u, mx_u = _tc_proj(xu, W_src[l, 0], att_src[l, 0],
                                   W_dst[l, 1], att_dst[l, 1])
        h_i, aa_i, mx_i = _tc_proj(xi, W_src[l, 1], att_src[l, 1],
                                   W_dst[l, 0], att_dst[l, 0])
        g0 = mx_u[0, 0] + mx_i[1, 0]
        g1 = mx_i[0, 0] + mx_u[1, 0]
        g0 = jnp.where(g0 >= 0.0, g0, 0.2 * g0)
        g1 = jnp.where(g1 >= 0.0, g1, 0.2 * g1)
        g2 = jnp.stack([jnp.full((16,), g0), jnp.full((16,), g1)])
        b2 = jnp.stack([bias[l, 0], bias[l, 1]])
        H = jnp.concatenate([h_u, h_i], axis=0)
        AS = jnp.concatenate([aa_u[:, 0], aa_i[:, 0]])
        AD = jnp.concatenate([aa_i[:, 1], aa_u[:, 1]])
        out = _sc_edge(H, AS, AD, srcg, dstl, g2, b2, zf, zv)
        xi = out[:NPAD]
        xu = out[NPAD:]
    return xu[:N], xi[:N]


# speedup vs baseline: 28.3097x; 1.0349x over previous
import functools

import jax
import jax.numpy as jnp
from jax import lax
from jax.experimental import pallas as pl
from jax.experimental.pallas import tpu as pltpu
from jax.experimental.pallas import tpu_sc as plsc

N = 10000
NPAD = 10240
D = 128
E = 160000
NCORE = 2
NSUB = 16
CH = 96
EPT = 10080
NCHUNK = EPT // CH
EPC = NSUB * EPT
RPT = NPAD // NSUB
WB = 32
BLK = 1024
NEG = -1e30


def _proj_body(x_ref, ws_ref, avs_ref, wd_ref, avd_ref, h_ref, aa_ref, mx_ref):
    x = x_ref[0]
    dn = (((1,), (1,)), ((), ()))
    ws_row = lax.dot_general(avs_ref[0], ws_ref[0], dn,
                             preferred_element_type=jnp.float32)
    wd_row = lax.dot_general(avd_ref[0], wd_ref[0], dn,
                             preferred_element_type=jnp.float32)
    h_ref[...] = jnp.dot(x, ws_ref[0], preferred_element_type=jnp.float32)
    a_s = jnp.sum(x * ws_row, axis=1, keepdims=True)
    a_d = jnp.sum(x * wd_row, axis=1, keepdims=True)
    aa_ref[...] = jnp.concatenate(
        [a_s, a_d, jnp.zeros((x.shape[0], 14), jnp.float32)], axis=1)
    cur = jnp.concatenate([
        jnp.full((1, 128), jnp.max(a_s), jnp.float32),
        jnp.full((1, 128), jnp.max(a_d), jnp.float32),
        jnp.full((6, 128), NEG, jnp.float32),
    ], axis=0)

    @pl.when(pl.program_id(1) == 0)
    def _():
        mx_ref[0] = cur

    @pl.when(pl.program_id(1) != 0)
    def _():
        mx_ref[0] = jnp.maximum(mx_ref[0], cur)


def _tc_proj(x2, Ws2, avs2, Wd2, avd2):
    return pl.pallas_call(
        _proj_body,
        grid=(2, NPAD // BLK),
        in_specs=[
            pl.BlockSpec((1, BLK, D), lambda t, i: (t, i, 0)),
            pl.BlockSpec((1, D, D), lambda t, i: (t, 0, 0)),
            pl.BlockSpec((1, 1, D), lambda t, i: (t, 0, 0)),
            pl.BlockSpec((1, D, D), lambda t, i: (t, 0, 0)),
            pl.BlockSpec((1, 1, D), lambda t, i: (t, 0, 0)),
        ],
        out_specs=[
            pl.BlockSpec((BLK, D), lambda t, i: (t * (NPAD // BLK) + i, 0)),
            pl.BlockSpec((BLK, 16), lambda t, i: (t * (NPAD // BLK) + i, 0)),
            pl.BlockSpec((1, 8, 128), lambda t, i: (t, 0, 0)),
        ],
        out_shape=[
            jax.ShapeDtypeStruct((2 * NPAD, D), jnp.float32),
            jax.ShapeDtypeStruct((2 * NPAD, 16), jnp.float32),
            jax.ShapeDtypeStruct((2, 8, 128), jnp.float32),
        ],
    )(x2.reshape(2, NPAD, D), Ws2, avs2.reshape(2, 1, D), Wd2,
      avd2.reshape(2, 1, D))


def _sc_edge(H, AS, AD, srcg, dstl, g2, b2, zf, zv):
    mesh = plsc.VectorSubcoreMesh(core_axis_name="c", subcore_axis_name="s",
                                  num_cores=NCORE, num_subcores=NSUB)

    @functools.partial(
        pl.kernel,
        out_type=jax.ShapeDtypeStruct((NCORE * NPAD, D), jnp.float32),
        mesh=mesh,
        compiler_params=pltpu.CompilerParams(needs_layout_passes=False),
        scratch_types=[
            pltpu.VMEM_SHARED((NPAD, D), jnp.float32),
            pltpu.VMEM_SHARED((NPAD,), jnp.float32),
            pltpu.VMEM((3, CH), jnp.int32),
            pltpu.VMEM((3, CH), jnp.int32),
            pltpu.VMEM((3, CH), jnp.int32),
            pltpu.VMEM((3, CH), jnp.float32),
            pltpu.VMEM((3, CH), jnp.float32),
            pltpu.VMEM((3, CH), jnp.float32),
            pltpu.VMEM((3, CH, D), jnp.float32),
            pltpu.VMEM((WB, D), jnp.float32),
            pltpu.VMEM((16,), jnp.float32),
            pltpu.VMEM((D,), jnp.float32),
            pltpu.VMEM((WB,), jnp.float32),
            pltpu.SemaphoreType.DMA,
            pltpu.SemaphoreType.DMA,
            pltpu.SemaphoreType.DMA,
            pltpu.SemaphoreType.DMA,
            pltpu.SemaphoreType.DMA,
            pltpu.SemaphoreType.DMA,
            pltpu.SemaphoreType.DMA,
            pltpu.SemaphoreType.DMA,
            pltpu.SemaphoreType.DMA,
        ],
    )
    def k(h_hbm, as_hbm, ad_hbm, src_hbm, dst_hbm, g_hbm, b_hbm, zf_hbm,
          zv_hbm, out, acc, dacc, idxs, idxd, idxg, asv, adv, pv, rows,
          nbuf, gv, bv, dnv, gs0, gs1, gs2, ss0, ss1, ss2, is0, is1, is2):
        c = lax.axis_index("c")
        s = lax.axis_index("s")
        gsem = (gs0, gs1, gs2)
        ssem = (ss0, ss1, ss2)
        isem = (is0, is1, is2)
        ro = s * RPT
        pltpu.sync_copy(zf_hbm.at[pl.ds(ro, RPT)], acc.at[pl.ds(ro, RPT)])
        pltpu.sync_copy(zv_hbm.at[pl.ds(ro, RPT)], dacc.at[pl.ds(ro, RPT)])
        pltpu.sync_copy(g_hbm.at[c], gv)
        pltpu.sync_copy(b_hbm.at[c], bv)
        plsc.subcore_barrier()
        ebase = c * EPC + s * EPT
        coff = (1 - c) * NPAD

        def starti(kk, b):
            off = pl.multiple_of(ebase + kk * CH, 8)
            pltpu.async_copy(src_hbm.at[pl.ds(off, CH)], idxs.at[b], isem[b])
            pltpu.async_copy(dst_hbm.at[pl.ds(off, CH)], idxd.at[b], isem[b])

        def waiti(b):
            pltpu.make_async_copy(src_hbm.at[pl.ds(0, CH)], idxs.at[b],
                                  isem[b]).wait()
            pltpu.make_async_copy(dst_hbm.at[pl.ds(0, CH)], idxd.at[b],
                                  isem[b]).wait()

        def startg(b):
            for j in range(CH // 16):
                sl = pl.ds(16 * j, 16)
                idxg[b, sl] = idxd[b, sl] + coff
            pltpu.async_copy(as_hbm.at[idxs.at[b]], asv.at[b], gsem[b])
            pltpu.async_copy(ad_hbm.at[idxg.at[b]], adv.at[b], gsem[b])
            pltpu.async_copy(h_hbm.at[idxs.at[b]], rows.at[b], gsem[b])

        def waitg(b):
            pltpu.make_async_copy(as_hbm.at[pl.ds(0, CH)], asv.at[b],
                                  gsem[b]).wait()
            pltpu.make_async_copy(ad_hbm.at[pl.ds(0, CH)], adv.at[b],
                                  gsem[b]).wait()
            pltpu.make_async_copy(h_hbm.at[pl.ds(0, CH)], rows.at[b],
                                  gsem[b]).wait()

        def starts(b):
            pltpu.async_copy(pv.at[b], dacc.at[idxd.at[b]], ssem[b],
                             add=True)
            pltpu.async_copy(rows.at[b], acc.at[idxd.at[b]], ssem[b],
                             add=True)

        def waits(b):
            pltpu.make_async_copy(zv_hbm.at[pl.ds(0, CH)], pv.at[b],
                                  ssem[b]).wait()
            pltpu.make_async_copy(zf_hbm.at[pl.ds(0, CH)], rows.at[b],
                                  ssem[b]).wait()

        starti(0, 0)
        waiti(0)
        startg(0)
        starti(1, 1)

        def ring(g, carry):
            k0 = 3 * g
            for b in range(3):
                kk = k0 + b
                bg = (b + 1) % 3
                bn = (b + 2) % 3
                waitg(b)

                @pl.when(kk + 1 < NCHUNK)
                def _(bg=bg):
                    waiti(bg)
                    startg(bg)

                gvec = gv[...]
                for j in range(CH // 16):
                    sl = pl.ds(16 * j, 16)
                    a = asv[b, sl] + adv[b, sl]
                    a = jnp.where(a >= 0.0, a, a * 0.2)
                    pv[b, sl] = jnp.exp(a - gvec)

                def srow(e, cc, b=b):
                    pe = plsc.load_gather(pv.at[b],
                                          [jnp.full((16,), e, jnp.int32)])
                    for j in range(8):
                        sl = pl.ds(16 * j, 16)
                        rows[b, e, sl] = rows[b, e, sl] * pe
                    return cc

                lax.fori_loop(0, CH, srow, 0, unroll=4)
                starts(b)
                kn = kk + 2

                @pl.when(kn < NCHUNK)
                def _(kk=kk, kn=kn, bn=bn):
                    @pl.when(kk >= 1)
                    def _():
                        waits(bn)

                    starti(kn, bn)

            return carry

        lax.fori_loop(0, NCHUNK // 3, ring, 0)
        waits(0)
        waits(1)
        waits(2)
        plsc.subcore_barrier()

        def wblock(b, carry):
            rbase = ro + b * WB
            pltpu.sync_copy(dacc.at[pl.ds(rbase, WB)], dnv)
            pltpu.sync_copy(acc.at[pl.ds(rbase, WB)], nbuf)

            def nrow(e, cc):
                d = plsc.load_gather(
                    dnv, [jnp.full((16,), e, jnp.int32)])
                q = 1.0 / (d + 1e-16)
                for j in range(8):
                    sl = pl.ds(16 * j, 16)
                    nbuf[e, sl] = jnp.maximum(nbuf[e, sl] * q + bv[sl], 0.0)
                return cc

            lax.fori_loop(0, WB, nrow, 0, unroll=4)
            pltpu.sync_copy(nbuf, out.at[pl.ds(coff + rbase, WB)])
            return carry

        lax.fori_loop(0, RPT // WB, wblock, 0)

    return k(H, AS, AD, srcg, dstl, g2, b2, zf, zv)


def kernel(x_user, x_item, edge_index_user_item, edge_index_item_user,
           W_src, W_dst, att_src, att_dst, bias):
    def _pad_edges(arr, fill):
        a = arr.astype(jnp.int32).reshape(NSUB, E // NSUB)
        a = jnp.pad(a, ((0, 0), (0, EPT - E // NSUB)), constant_values=fill)
        return a.reshape(EPC)

    src0 = _pad_edges(edge_index_user_item[0], 0)
    dst0 = _pad_edges(edge_index_user_item[1], NPAD - 1)
    src1 = _pad_edges(edge_index_item_user[0], 0)
    dst1 = _pad_edges(edge_index_item_user[1], NPAD - 1)
    srcg = jnp.concatenate([src0, src1 + NPAD])
    dstl = jnp.concatenate([dst0, dst1])
    pad = ((0, NPAD - N), (0, 0))
    x2 = jnp.concatenate([jnp.pad(x_user, pad), jnp.pad(x_item, pad)], axis=0)
    zf = jnp.zeros((NPAD, D), jnp.float32)
    zv = jnp.zeros((NPAD,), jnp.float32)
    for l in range(2):
        H, aa, mx = _tc_proj(x2, W_src[l], att_src[l],
                             W_dst[l, ::-1], att_dst[l, ::-1])
        g0 = mx[0, 0, 0] + mx[1, 1, 0]
        g1 = mx[1, 0, 0] + mx[0, 1, 0]
        g0 = jnp.where(g0 >= 0.0, g0, 0.2 * g0)
        g1 = jnp.where(g1 >= 0.0, g1, 0.2 * g1)
        g2 = jnp.stack([jnp.full((16,), g0), jnp.full((16,), g1)])
        b2 = jnp.stack([bias[l, 0], bias[l, 1]])
        x2 = _sc_edge(H, aa[:, 0], aa[:, 1], srcg, dstl, g2, b2, zf, zv)
    return x2[:N], x2[NPAD:NPAD + N]
